# baseline (speedup 1.0000x reference)
import jax
import jax.numpy as jnp
from jax.experimental import pallas as pl
from jax.experimental.pallas import tpu as pltpu

N_TOKENS = 32768
D_MODEL = 768
N_EXP = 8
BLK = 2048


def _router_blk(x_ref, w_ref, logits_ref, probs_ref, ent_ref):
    i = pl.program_id(0)
    x = x_ref[...]
    w = w_ref[...]
    logits = jax.lax.dot_general(
        x, w, (((1,), (1,)), ((), ())),
        preferred_element_type=jnp.float32)
    m = jnp.max(logits, axis=-1, keepdims=True)
    e = jnp.exp(logits - m)
    s = jnp.sum(e, axis=-1, keepdims=True)
    probs = e / s
    logits_ref[...] = logits
    probs_ref[...] = probs
    ent_blk = -jnp.sum(probs * jnp.log(probs + 1e-8))

    @pl.when(i == 0)
    def _init():
        ent_ref[0, 0] = ent_blk

    @pl.when(i != 0)
    def _acc():
        ent_ref[0, 0] += ent_blk


def kernel(x, W):
    grid = N_TOKENS // BLK
    logits, probs, ent_sum = pl.pallas_call(
        _router_blk,
        grid=(grid,),
        in_specs=[
            pl.BlockSpec((BLK, D_MODEL), lambda i: (i, 0)),
            pl.BlockSpec((N_EXP, D_MODEL), lambda i: (0, 0)),
        ],
        out_specs=[
            pl.BlockSpec((BLK, N_EXP), lambda i: (i, 0)),
            pl.BlockSpec((BLK, N_EXP), lambda i: (i, 0)),
            pl.BlockSpec(memory_space=pltpu.SMEM, block_shape=(1, 1),
                         index_map=lambda i: (0, 0)),
        ],
        out_shape=[
            jax.ShapeDtypeStruct((N_TOKENS, N_EXP), jnp.float32),
            jax.ShapeDtypeStruct((N_TOKENS, N_EXP), jnp.float32),
            jax.ShapeDtypeStruct((1, 1), jnp.float32),
        ],
    )(x, W)
    router_entropy = ent_sum[0, 0] / N_TOKENS
    return (logits, probs, router_entropy)
